# Initial kernel scaffold; baseline (speedup 1.0000x reference)
import jax
import jax.numpy as jnp
from jax.experimental import pallas as pl


def kernel(decimal_tensor, B):
    raise NotImplementedError("write your pallas kernel here")


# speedup vs baseline: 3.3363x; 3.3363x over previous
import functools

import jax
import jax.numpy as jnp
from jax import lax
from jax.experimental import pallas as pl
from jax.experimental.pallas import tpu as pltpu
from jax.experimental.pallas import tpu_sc as plsc

K = 16
OB = 4
CH = 2048
GROUPS = CH // 16


def _make_sc_call(n_rows: int):
    info = plsc.get_sparse_core_info()
    nw = info.num_cores * info.num_subcores
    rows_w = n_rows // nw
    assert rows_w * nw == n_rows and rows_w % CH == 0
    nchunk = rows_w // CH

    mesh = plsc.VectorSubcoreMesh(core_axis_name="c", subcore_axis_name="s")

    @functools.partial(
        pl.kernel,
        out_type=jax.ShapeDtypeStruct((n_rows * OB,), jnp.float32),
        mesh=mesh,
        scratch_types=[
            pltpu.VMEM((CH * K,), jnp.float32),
            pltpu.VMEM((CH * K,), jnp.float32),
            pltpu.VMEM((CH * OB,), jnp.float32),
            pltpu.VMEM((CH * OB,), jnp.float32),
            pltpu.VMEM((K * OB,), jnp.float32),
            pltpu.SemaphoreType.DMA,
            pltpu.SemaphoreType.DMA,
            pltpu.SemaphoreType.DMA,
            pltpu.SemaphoreType.DMA,
        ],
        compiler_params=pltpu.CompilerParams(needs_layout_passes=False),
    )
    def sc_kernel(x_hbm, b_hbm, out_hbm, in0, in1, out0, out1, bv,
                  isem0, isem1, osem0, osem1):
        wid = lax.axis_index("s") * info.num_cores + lax.axis_index("c")
        row0 = wid * rows_w

        inbufs, insems = (in0, in1), (isem0, isem1)
        outbufs, outsems = (out0, out1), (osem0, osem1)

        pltpu.sync_copy(b_hbm, bv)

        def copy_in(ci, buf, sem):
            return pltpu.async_copy(
                x_hbm.at[pl.ds((row0 + ci * CH) * K, CH * K)], buf, sem)

        def copy_out(ci, buf, sem):
            return pltpu.async_copy(
                buf, out_hbm.at[pl.ds((row0 + ci * CH) * OB, CH * OB)], sem)

        iota = lax.iota(jnp.int32, K)
        stride = iota * K
        st_stride = iota * OB

        def compute(in_ref, out_ref):
            def group(g, carry):
                gw = g * (16 * K)
                ow = g * (16 * OB)
                col0 = gw + stride
                m = plsc.load_gather(in_ref, [col0])
                idxv = jnp.zeros((K,), jnp.int32)
                for c in range(1, K):
                    v = plsc.load_gather(in_ref, [col0 + c])
                    pred = v > m
                    m = jnp.where(pred, v, m)
                    idxv = jnp.where(pred, c, idxv)
                g4 = idxv * OB
                for j in range(OB):
                    o = plsc.load_gather(bv, [g4 + j])
                    plsc.store_scatter(out_ref, [ow + st_stride + j], o)
                return carry
            lax.fori_loop(0, GROUPS, group, 0)

        in_h = [copy_in(0, in0, isem0), None]
        if nchunk > 1:
            in_h[1] = copy_in(1, in1, isem1)
        out_h = [None, None]
        for ci in range(nchunk):
            b = ci % 2
            in_h[b].wait()
            if out_h[b] is not None:
                out_h[b].wait()
            compute(inbufs[b], outbufs[b])
            out_h[b] = copy_out(ci, outbufs[b], outsems[b])
            if ci + 2 < nchunk:
                in_h[b] = copy_in(ci + 2, inbufs[b], insems[b])
        for b in range(2):
            if out_h[b] is not None:
                out_h[b].wait()

    return sc_kernel


@jax.jit
def kernel(decimal_tensor, B):
    n = decimal_tensor.shape[0]
    out = _make_sc_call(n)(decimal_tensor.reshape(-1), B.reshape(-1))
    return out.reshape(n, 1, OB)


# speedup vs baseline: 3.3855x; 1.0148x over previous
import functools

import jax
import jax.numpy as jnp
from jax import lax
from jax.experimental import pallas as pl
from jax.experimental.pallas import tpu as pltpu
from jax.experimental.pallas import tpu_sc as plsc

K = 16
OB = 4
CH = 2048
GROUPS = CH // 16


def _make_sc_call(n_rows: int):
    info = plsc.get_sparse_core_info()
    nw = info.num_cores * info.num_subcores
    rows_w = n_rows // nw
    assert rows_w * nw == n_rows and rows_w % CH == 0
    nchunk = rows_w // CH

    mesh = plsc.VectorSubcoreMesh(core_axis_name="c", subcore_axis_name="s")

    @functools.partial(
        pl.kernel,
        out_type=jax.ShapeDtypeStruct((n_rows * OB,), jnp.float32),
        mesh=mesh,
        scratch_types=[
            pltpu.VMEM((CH * K,), jnp.float32),
            pltpu.VMEM((CH * K,), jnp.float32),
            pltpu.VMEM((CH * OB,), jnp.float32),
            pltpu.VMEM((CH * OB,), jnp.float32),
            pltpu.VMEM((K * OB,), jnp.float32),
            pltpu.SemaphoreType.DMA,
            pltpu.SemaphoreType.DMA,
            pltpu.SemaphoreType.DMA,
            pltpu.SemaphoreType.DMA,
        ],
        compiler_params=pltpu.CompilerParams(needs_layout_passes=False),
    )
    def sc_kernel(x_hbm, b_hbm, out_hbm, in0, in1, out0, out1, bv,
                  isem0, isem1, osem0, osem1):
        wid = lax.axis_index("s") * info.num_cores + lax.axis_index("c")
        row0 = wid * rows_w

        inbufs, insems = (in0, in1), (isem0, isem1)
        outbufs, outsems = (out0, out1), (osem0, osem1)

        pltpu.sync_copy(b_hbm, bv)

        def copy_in(ci, buf, sem):
            return pltpu.async_copy(
                x_hbm.at[pl.ds((row0 + ci * CH) * K, CH * K)], buf, sem)

        def copy_out(ci, buf, sem):
            return pltpu.async_copy(
                buf, out_hbm.at[pl.ds((row0 + ci * CH) * OB, CH * OB)], sem)

        iota = lax.iota(jnp.int32, K)
        stride = iota * K
        st_stride = iota * OB

        idx_consts = [jnp.full((K,), c, jnp.int32) for c in range(K)]

        def compute(in_ref, out_ref):
            @plsc.parallel_loop(0, GROUPS, 1, unroll=4)
            def _group(g):
                col0 = g * (16 * K) + stride
                ms = [plsc.load_gather(in_ref, [col0 + c]) for c in range(K)]
                ixs = idx_consts
                while len(ms) > 1:
                    nm, ni = [], []
                    for a in range(0, len(ms), 2):
                        pred = ms[a + 1] > ms[a]
                        nm.append(jnp.where(pred, ms[a + 1], ms[a]))
                        ni.append(jnp.where(pred, ixs[a + 1], ixs[a]))
                    ms, ixs = nm, ni
                g4 = ixs[0] * OB
                ow = g * (16 * OB) + st_stride
                for j in range(OB):
                    o = plsc.load_gather(bv, [g4 + j])
                    plsc.store_scatter(out_ref, [ow + j], o)

        in_h = [copy_in(0, in0, isem0), None]
        if nchunk > 1:
            in_h[1] = copy_in(1, in1, isem1)
        out_h = [None, None]
        for ci in range(nchunk):
            b = ci % 2
            in_h[b].wait()
            if out_h[b] is not None:
                out_h[b].wait()
            compute(inbufs[b], outbufs[b])
            out_h[b] = copy_out(ci, outbufs[b], outsems[b])
            if ci + 2 < nchunk:
                in_h[b] = copy_in(ci + 2, inbufs[b], insems[b])
        for b in range(2):
            if out_h[b] is not None:
                out_h[b].wait()

    return sc_kernel


@jax.jit
def kernel(decimal_tensor, B):
    n = decimal_tensor.shape[0]
    out = _make_sc_call(n)(decimal_tensor.reshape(-1), B.reshape(-1))
    return out.reshape(n, 1, OB)
